# Initial kernel scaffold; baseline (speedup 1.0000x reference)
import jax
import jax.numpy as jnp
from jax.experimental import pallas as pl


def kernel(x, table):
    raise NotImplementedError("write your pallas kernel here")


# speedup vs baseline: 4.6995x; 4.6995x over previous
import functools

import jax
import jax.numpy as jnp
from jax import lax
from jax.experimental import pallas as pl
from jax.experimental.pallas import tpu as pltpu
from jax.experimental.pallas import tpu_sc as plsc

_ROWS, _COLS = 16384, 200
_B = _ROWS * _COLS
_D = 16
_V = 64
_NC, _NS = 2, 16
_NW = _NC * _NS
_BPW = _B // _NW
_CH = 2048
_NG = _CH // 16
_NCH = _BPW // _CH


def _make_emb():
    mesh = plsc.VectorSubcoreMesh(core_axis_name="c", subcore_axis_name="s")

    @functools.partial(
        pl.kernel,
        mesh=mesh,
        compiler_params=pltpu.CompilerParams(needs_layout_passes=False),
        out_type=jax.ShapeDtypeStruct((_B * _D,), jnp.float32),
        scratch_types=[
            pltpu.VMEM((_V * _D,), jnp.float32),
            pltpu.VMEM((_CH,), jnp.int32),
            pltpu.VMEM((_CH * _D,), jnp.float32),
        ],
    )
    def emb(idx_hbm, table_hbm, out_hbm, tab_v, idx_v, rows_v):
        wid = lax.axis_index("s") * _NC + lax.axis_index("c")
        wbase = wid * _BPW
        pltpu.sync_copy(table_hbm, tab_v)
        iota = lax.iota(jnp.int32, 16)
        iota16 = iota * _D

        def chunk_body(c, carry):
            base = wbase + c * _CH
            pltpu.sync_copy(idx_hbm.at[pl.ds(base, _CH)], idx_v)

            def group_body(g, carry2):
                ivec = idx_v[pl.ds(g * 16, 16)]
                ivec16 = ivec * _D
                gbase = g * (16 * _D)
                for d in range(_D):
                    vals = plsc.load_gather(tab_v, [ivec16 + d])
                    plsc.store_scatter(rows_v, [iota16 + (gbase + d)], vals)
                return carry2

            lax.fori_loop(0, _NG, group_body, 0)
            pltpu.sync_copy(rows_v, out_hbm.at[pl.ds(base * _D, _CH * _D)])
            return carry

        lax.fori_loop(0, _NCH, chunk_body, 0)

    return emb


_emb = _make_emb()


def kernel(x, table):
    flat = _emb(x.reshape(_B), table.reshape(_V * _D))
    return flat.reshape(_ROWS, _COLS, _D)


# speedup vs baseline: 5.5979x; 1.1912x over previous
import functools

import jax
import jax.numpy as jnp
from jax import lax
from jax.experimental import pallas as pl
from jax.experimental.pallas import tpu as pltpu
from jax.experimental.pallas import tpu_sc as plsc

_ROWS, _COLS = 16384, 200
_B = _ROWS * _COLS
_D = 16
_V = 64
_NC, _NS = 2, 16
_NW = _NC * _NS
_BPW = _B // _NW
_CH = 2048
_NG = _CH // 16
_NCH = _BPW // _CH


def _make_emb():
    mesh = plsc.VectorSubcoreMesh(core_axis_name="c", subcore_axis_name="s")

    @functools.partial(
        pl.kernel,
        mesh=mesh,
        compiler_params=pltpu.CompilerParams(needs_layout_passes=False),
        out_type=jax.ShapeDtypeStruct((_B * _D,), jnp.float32),
        scratch_types=[
            pltpu.VMEM((_V * _D,), jnp.float32),
            pltpu.VMEM((_CH,), jnp.int32),
            pltpu.VMEM((_CH,), jnp.int32),
            pltpu.VMEM((_CH * _D,), jnp.float32),
            pltpu.VMEM((_CH * _D,), jnp.float32),
            pltpu.SemaphoreType.DMA,
            pltpu.SemaphoreType.DMA,
            pltpu.SemaphoreType.DMA,
            pltpu.SemaphoreType.DMA,
        ],
    )
    def emb(idx_hbm, table_hbm, out_hbm,
            tab_v, idx0, idx1, rows0, rows1, sin0, sin1, sout0, sout1):
        wid = lax.axis_index("s") * _NC + lax.axis_index("c")
        wbase = wid * _BPW
        pltpu.sync_copy(table_hbm, tab_v)
        iota16 = lax.iota(jnp.int32, 16) * _D
        idx_b = (idx0, idx1)
        rows_b = (rows0, rows1)
        sin_b = (sin0, sin1)
        sout_b = (sout0, sout1)

        def idx_src(c):
            return idx_hbm.at[pl.ds(wbase + c * _CH, _CH)]

        def out_dst(c):
            return out_hbm.at[pl.ds((wbase + c * _CH) * _D, _CH * _D)]

        pltpu.async_copy(idx_src(0), idx0, sin0)
        pltpu.async_copy(idx_src(1), idx1, sin1)

        def chunk_pair(i, carry):
            cc = i * 2
            for b in range(2):
                c = cc + b
                idxv, rowsv = idx_b[b], rows_b[b]
                pltpu.make_async_copy(idx_src(c), idxv, sin_b[b]).wait()

                @pl.when(c >= 2)
                def _():
                    pltpu.make_async_copy(rowsv, out_dst(c - 2),
                                          sout_b[b]).wait()

                @plsc.parallel_loop(0, _NG, unroll=4)
                def _group(g):
                    ivec16 = idxv[pl.ds(g * 16, 16)] * _D
                    pos0 = iota16 + g * (16 * _D)
                    for d in range(_D):
                        vals = plsc.load_gather(tab_v, [ivec16 + d])
                        plsc.store_scatter(rowsv, [pos0 + d], vals)

                pltpu.async_copy(rowsv, out_dst(c), sout_b[b])

                @pl.when(c + 2 < _NCH)
                def _():
                    pltpu.async_copy(idx_src(c + 2), idxv, sin_b[b])
            return carry

        lax.fori_loop(0, _NCH // 2, chunk_pair, 0)
        for b in range(2):
            pltpu.make_async_copy(rows_b[b], out_dst(_NCH - 2 + b),
                                  sout_b[b]).wait()

    return emb


_emb = _make_emb()


def kernel(x, table):
    flat = _emb(x.reshape(_B), table.reshape(_V * _D))
    return flat.reshape(_ROWS, _COLS, _D)


# speedup vs baseline: 6.9538x; 1.2422x over previous
import functools

import jax
import jax.numpy as jnp
from jax import lax
from jax.experimental import pallas as pl
from jax.experimental.pallas import tpu as pltpu
from jax.experimental.pallas import tpu_sc as plsc

_ROWS, _COLS = 16384, 200
_B = _ROWS * _COLS
_D = 16
_V = 64
_NC, _NS = 2, 16
_NW = _NC * _NS
_BPW = _B // _NW
_CH = 2048
_NG = _CH // 16
_NCH = _BPW // _CH


def _make_emb():
    mesh = plsc.VectorSubcoreMesh(core_axis_name="c", subcore_axis_name="s")

    @functools.partial(
        pl.kernel,
        mesh=mesh,
        compiler_params=pltpu.CompilerParams(
            needs_layout_passes=False, disable_bounds_checks=True),
        out_type=jax.ShapeDtypeStruct((_B * _D,), jnp.float32),
        scratch_types=[
            pltpu.VMEM((_V * _D,), jnp.float32),
            pltpu.VMEM((_CH,), jnp.int32),
            pltpu.VMEM((_CH,), jnp.int32),
            pltpu.VMEM((_CH * _D,), jnp.float32),
            pltpu.VMEM((_CH * _D,), jnp.float32),
            pltpu.SemaphoreType.DMA,
            pltpu.SemaphoreType.DMA,
            pltpu.SemaphoreType.DMA,
            pltpu.SemaphoreType.DMA,
        ],
    )
    def emb(idx_hbm, table_hbm, out_hbm,
            tab_v, idx0, idx1, rows0, rows1, sin0, sin1, sout0, sout1):
        wid = lax.axis_index("s") * _NC + lax.axis_index("c")
        wbase = wid * _BPW
        pltpu.sync_copy(table_hbm, tab_v)
        iota16 = lax.iota(jnp.int32, 16) * _D
        idx_b = (idx0, idx1)
        rows_b = (rows0, rows1)
        sin_b = (sin0, sin1)
        sout_b = (sout0, sout1)

        def idx_src(c):
            return idx_hbm.at[pl.ds(wbase + c * _CH, _CH)]

        def out_dst(c):
            return out_hbm.at[pl.ds((wbase + c * _CH) * _D, _CH * _D)]

        pltpu.async_copy(idx_src(0), idx0, sin0)
        pltpu.async_copy(idx_src(1), idx1, sin1)

        def chunk_pair(i, carry):
            cc = i * 2
            for b in range(2):
                c = cc + b
                idxv, rowsv = idx_b[b], rows_b[b]
                pltpu.make_async_copy(idx_src(c), idxv, sin_b[b]).wait()

                @pl.when(c >= 2)
                def _():
                    pltpu.make_async_copy(rowsv, out_dst(c - 2),
                                          sout_b[b]).wait()

                @plsc.parallel_loop(0, _NG, unroll=2)
                def _group(g):
                    ivec = idxv[pl.ds(g * 16, 16)]
                    gbase = g * (16 * _D)
                    for k in range(16):
                        s = ivec[k]
                        rowsv[pl.ds(gbase + k * _D, _D)] = (
                            tab_v[pl.ds(s * _D, _D)])

                pltpu.async_copy(rowsv, out_dst(c), sout_b[b])

                @pl.when(c + 2 < _NCH)
                def _():
                    pltpu.async_copy(idx_src(c + 2), idxv, sin_b[b])
            return carry

        lax.fori_loop(0, _NCH // 2, chunk_pair, 0)
        for b in range(2):
            pltpu.make_async_copy(rows_b[b], out_dst(_NCH - 2 + b),
                                  sout_b[b]).wait()

    return emb


_emb = _make_emb()


def kernel(x, table):
    flat = _emb(x.reshape(_B), table.reshape(_V * _D))
    return flat.reshape(_ROWS, _COLS, _D)
